# baseline (speedup 1.0000x reference)
import functools

import jax
import jax.numpy as jnp
from jax import lax
from jax.experimental import pallas as pl
from jax.experimental.pallas import tpu as pltpu
from jax.experimental.pallas import tpu_sc as plsc

_BATCH = 16384
_FIELDS = 26
_DIM = 64
_B = _BATCH * _FIELDS
_NC = 2
_NS = 16
_NW = _NC * _NS
_CHUNK = 128
_ROWS_PER_W = _B // _NW
_CPW = _ROWS_PER_W // _CHUNK

_mesh = plsc.VectorSubcoreMesh(core_axis_name="c", subcore_axis_name="s")


@functools.partial(
    pl.kernel,
    mesh=_mesh,
    compiler_params=pltpu.CompilerParams(use_tc_tiling_on_sc=False),
    out_type=jax.ShapeDtypeStruct((_B, _DIM), jnp.float32),
    scratch_types=[
        pltpu.VMEM((_CPW, _CHUNK), jnp.int32),
        pltpu.VMEM((_CHUNK, _DIM), jnp.float32),
        pltpu.SemaphoreType.DMA,
    ],
)
def _emb_gather(idx_hbm, table_hbm, out_hbm, idx_v, rows_v, sem):
    wid = lax.axis_index("s") * _NC + lax.axis_index("c")
    pltpu.sync_copy(idx_hbm.at[pl.ds(wid * _CPW, _CPW)], idx_v)

    def body(j, carry):
        pltpu.async_copy(table_hbm.at[idx_v.at[j]], rows_v, sem).wait()
        pltpu.sync_copy(
            rows_v, out_hbm.at[pl.ds((wid * _CPW + j) * _CHUNK, _CHUNK)]
        )
        return carry

    lax.fori_loop(0, _CPW, body, 0)


def kernel(input, weight):
    idx = input.reshape(_B // _CHUNK, _CHUNK).astype(jnp.int32)
    out = _emb_gather(idx, weight)
    return out.reshape(_BATCH, _FIELDS, _DIM)


# speedup vs baseline: 1.3752x; 1.3752x over previous
import functools

import jax
import jax.numpy as jnp
from jax import lax
from jax.experimental import pallas as pl
from jax.experimental.pallas import tpu as pltpu
from jax.experimental.pallas import tpu_sc as plsc

_BATCH = 16384
_FIELDS = 26
_DIM = 64
_B = _BATCH * _FIELDS
_NC = 2
_NS = 16
_NW = _NC * _NS
_CHUNK = 128
_ROWS_PER_W = _B // _NW
_CPW = _ROWS_PER_W // _CHUNK

_NBUF = 8

_mesh = plsc.VectorSubcoreMesh(core_axis_name="c", subcore_axis_name="s")


@functools.partial(
    pl.kernel,
    mesh=_mesh,
    compiler_params=pltpu.CompilerParams(use_tc_tiling_on_sc=False),
    out_type=jax.ShapeDtypeStruct((_B, _DIM), jnp.float32),
    scratch_types=[
        pltpu.VMEM((_CPW, _CHUNK), jnp.int32),
        [pltpu.VMEM((_CHUNK, _DIM), jnp.float32) for _ in range(_NBUF)],
        pltpu.SemaphoreType.DMA((_NBUF,)),
    ],
)
def _emb_gather(idx_hbm, table_hbm, out_hbm, idx_v, rows, sems):
    wid = lax.axis_index("s") * _NC + lax.axis_index("c")
    pltpu.sync_copy(idx_hbm.at[pl.ds(wid * _CPW, _CPW)], idx_v)

    def fire(j, b):
        pltpu.async_copy(table_hbm.at[idx_v.at[j]], rows[b], sems.at[b])

    def drain(j, b):
        pltpu.make_async_copy(
            table_hbm.at[idx_v.at[j]], rows[b], sems.at[b]
        ).wait()
        pltpu.sync_copy(
            rows[b], out_hbm.at[pl.ds((wid * _CPW + j) * _CHUNK, _CHUNK)]
        )

    for b in range(_NBUF):
        fire(b, b)

    def group(g, carry):
        for b in range(_NBUF):
            j = g * _NBUF + b
            drain(j, b)
            nxt = j + _NBUF

            @pl.when(nxt < _CPW)
            def _():
                fire(nxt, b)

        return carry

    lax.fori_loop(0, _CPW // _NBUF, group, 0)


def kernel(input, weight):
    idx = input.reshape(_B // _CHUNK, _CHUNK).astype(jnp.int32)
    out = _emb_gather(idx, weight)
    return out.reshape(_BATCH, _FIELDS, _DIM)
